# baseline (speedup 1.0000x reference)
import jax
import jax.numpy as jnp
from jax import lax
from jax.experimental import pallas as pl
from jax.experimental.pallas import tpu as pltpu
from jax.experimental.pallas import tpu_sc as plsc

N = 10000
D = 256
E = 160000
NC = 2
NS = 16
NW = NC * NS
L = 16

RPT = 320
LASTR = N - (NW - 1) * RPT
ACC_ROWS = RPT + 1
DUMP = RPT
CHUNK = 1600
NCH = E // CHUNK
U = 2
G = 64
CAP = CHUNK + G
NBMAX = CAP // G + 1


def _sc_body(x_hbm, src_hbm, dst_hbm, out_hbm,
             srcb0, dstb0, srcb1, dstb1, src_c, ldst_c,
             snap_src_a, snap_ld_a, snap_src_b, snap_ld_b,
             rows_a, rows_b, acc,
             gsem_a, gsem_b, esem0, esem1):
  c = lax.axis_index("c")
  s = lax.axis_index("s")
  w = c * NS + s
  lo = w * RPT
  hi = jnp.minimum(lo + RPT, N)
  zero_v = jnp.zeros((L,), jnp.float32)
  ones_i = jnp.ones((L,), jnp.int32)
  zeros_i = jnp.zeros((L,), jnp.int32)
  lane = lax.iota(jnp.int32, L)

  def _zrow(i, _):
    def _zcol(k, _):
      acc[i, pl.ds(k * L, L)] = zero_v
      return 0
    return lax.fori_loop(0, D // L, _zcol, 0)
  lax.fori_loop(0, ACC_ROWS, _zrow, 0)

  def fire(off, snap_src, snap_ld, rows, gsem):
    for k in range(G // L):
      snap_src[pl.ds(k * L, L)] = src_c[pl.ds(off + k * L, L)]
      snap_ld[pl.ds(k * L, L)] = ldst_c[pl.ds(off + k * L, L)]
    pltpu.async_copy(x_hbm.at[snap_src], rows, gsem)

  def process(snap_src, snap_ld, rows, gsem):
    pltpu.make_async_copy(x_hbm.at[snap_src], rows, gsem).wait()

    def _rowgrp(gg, _):
      ldv = snap_ld[pl.ds(gg * L, L)]
      for j in range(L):
        ld = ldv[j]
        g = gg * L + j
        for k in range(D // L):
          plsc.addupdate(acc.at[ld, pl.ds(k * L, L)],
                         rows[g, pl.ds(k * L, L)])
      return 0
    lax.fori_loop(0, G // L, _rowgrp, 0)

  SLOT_A = (snap_src_a, snap_ld_a, rows_a, gsem_a)
  SLOT_B = (snap_src_b, snap_ld_b, rows_b, gsem_b)

  def fire_process(off, bk):
    even = bk % 2 == 0

    @pl.when(even)
    def _():
      fire(off, *SLOT_A)

      @pl.when(bk >= 1)
      def _():
        process(*SLOT_B)

    @pl.when(jnp.logical_not(even))
    def _():
      fire(off, *SLOT_B)
      process(*SLOT_A)

  def start_chunk(ci, sb, db, sem):
    pltpu.async_copy(src_hbm.at[pl.ds(ci * CHUNK, CHUNK)], sb, sem)
    pltpu.async_copy(dst_hbm.at[pl.ds(ci * CHUNK, CHUNK)], db, sem)

  def wait_chunk(ci, sb, db, sem):
    pltpu.make_async_copy(src_hbm.at[pl.ds(ci * CHUNK, CHUNK)], sb, sem).wait()
    pltpu.make_async_copy(dst_hbm.at[pl.ds(ci * CHUNK, CHUNK)], db, sem).wait()

  def scan_chunk(sb, db, cur0):
    def _scan(i, cur):
      regs = []
      for u in range(U):
        sv = sb[pl.ds((i * U + u) * L, L)]
        dv = db[pl.ds((i * U + u) * L, L)]
        ld = dv - lo
        m = (ld >= 0) & (dv < hi)
        mi = jnp.where(m, ones_i, zeros_i)
        regs.append((sv, ld, m, plsc.cumsum(mi)))
      for sv, ld, m, cs in regs:
        pos = cur + cs - 1
        plsc.store_scatter(src_c, [pos], sv, mask=m)
        plsc.store_scatter(ldst_c, [pos], ld, mask=m)
        cur = cur + cs[L - 1]
      return cur
    return lax.fori_loop(0, CHUNK // (U * L), _scan, cur0)

  def drain(cur, bk):
    def _batch(b, bk):
      @pl.when((b + 1) * G <= cur)
      def _():
        fire_process(b * G, bk)
      return jnp.where((b + 1) * G <= cur, bk + 1, bk)
    bk = lax.fori_loop(0, NBMAX, _batch, bk)

    off0 = (cur // G) * G
    for k in range(G // L):
      sv = src_c[pl.ds(off0 + k * L, L)]
      lv = ldst_c[pl.ds(off0 + k * L, L)]
      src_c[pl.ds(k * L, L)] = sv
      ldst_c[pl.ds(k * L, L)] = lv
    return cur - off0, bk

  start_chunk(0, srcb0, dstb0, esem0)

  def _pair(ci2, carry):
    cur, bk = carry
    ci = ci2 * 2

    @pl.when(ci + 1 < NCH)
    def _():
      start_chunk(ci + 1, srcb1, dstb1, esem1)
    wait_chunk(ci, srcb0, dstb0, esem0)
    cur, bk = drain(scan_chunk(srcb0, dstb0, cur), bk)

    @pl.when(ci + 2 < NCH)
    def _():
      start_chunk(ci + 2, srcb0, dstb0, esem0)
    wait_chunk(ci + 1, srcb1, dstb1, esem1)
    cur, bk = drain(scan_chunk(srcb1, dstb1, cur), bk)
    return cur, bk

  rem, bk = lax.fori_loop(0, NCH // 2, _pair, (jnp.int32(0), jnp.int32(0)))

  @pl.when(rem > 0)
  def _():
    for k in range(G // L):
      pos = rem + (k * L) + lane
      plsc.store_scatter(src_c, [pos], zeros_i)
      plsc.store_scatter(ldst_c, [pos], jnp.full((L,), DUMP, jnp.int32))
    fire_process(0, bk)
  bk = jnp.where(rem > 0, bk + 1, bk)

  @pl.when(bk >= 1)
  def _():
    last_even = (bk - 1) % 2 == 0

    @pl.when(last_even)
    def _():
      process(*SLOT_A)

    @pl.when(jnp.logical_not(last_even))
    def _():
      process(*SLOT_B)

  @pl.when(w < NW - 1)
  def _():
    pltpu.sync_copy(acc.at[pl.ds(0, RPT)], out_hbm.at[pl.ds(lo, RPT)])

  @pl.when(w == NW - 1)
  def _():
    pltpu.sync_copy(acc.at[pl.ds(0, LASTR)], out_hbm.at[pl.ds(lo, LASTR)])


def _sc_agg(x, src, dst):
  mesh = plsc.VectorSubcoreMesh(core_axis_name="c", subcore_axis_name="s")
  kern = pl.kernel(
      _sc_body,
      out_type=jax.ShapeDtypeStruct((N, D), jnp.float32),
      mesh=mesh,
      compiler_params=pltpu.CompilerParams(needs_layout_passes=False),
      scratch_types=[
          pltpu.VMEM((CHUNK,), jnp.int32),
          pltpu.VMEM((CHUNK,), jnp.int32),
          pltpu.VMEM((CHUNK,), jnp.int32),
          pltpu.VMEM((CHUNK,), jnp.int32),
          pltpu.VMEM((CAP,), jnp.int32),
          pltpu.VMEM((CAP,), jnp.int32),
          pltpu.VMEM((G,), jnp.int32),
          pltpu.VMEM((G,), jnp.int32),
          pltpu.VMEM((G,), jnp.int32),
          pltpu.VMEM((G,), jnp.int32),
          pltpu.VMEM((G, D), jnp.float32),
          pltpu.VMEM((G, D), jnp.float32),
          pltpu.VMEM((ACC_ROWS, D), jnp.float32),
          pltpu.SemaphoreType.DMA,
          pltpu.SemaphoreType.DMA,
          pltpu.SemaphoreType.DMA,
          pltpu.SemaphoreType.DMA,
      ],
  )
  return kern(x, src, dst)


def _bf16x3_dot(a, b):
  ah = a.astype(jnp.bfloat16)
  al = (a - ah.astype(jnp.float32)).astype(jnp.bfloat16)
  bh = b.astype(jnp.bfloat16)
  bl = (b - bh.astype(jnp.float32)).astype(jnp.bfloat16)
  def d(x, y):
    return jnp.dot(x, y, preferred_element_type=jnp.float32)
  return d(ah, bh) + d(ah, bl) + d(al, bh)


def _dense_body(agg_ref, w1_ref, b1_ref, w2_ref, b2_ref, out_ref):
  h = jnp.maximum(_bf16x3_dot(agg_ref[...], w1_ref[...]) + b1_ref[...], 0.0)
  pooled = jnp.sum(h, axis=0, keepdims=True)
  out_ref[...] = _bf16x3_dot(pooled, w2_ref[...]) + b2_ref[...]


def _dense(agg, W1, b1, W2, b2):
  return pl.pallas_call(
      _dense_body,
      out_shape=jax.ShapeDtypeStruct((1, 1), jnp.float32),
  )(agg, W1, b1.reshape(1, D), W2, b2.reshape(1, 1))


@jax.jit
def kernel(x, edge_index, W1, b1, W2, b2):
  src = edge_index[0].astype(jnp.int32)
  dst = edge_index[1].astype(jnp.int32)
  agg = _sc_agg(x, src, dst)
  return _dense(agg, W1, b1, W2, b2)


# speedup vs baseline: 1.2675x; 1.2675x over previous
import jax
import jax.numpy as jnp
from jax import lax
from jax.experimental import pallas as pl
from jax.experimental.pallas import tpu as pltpu
from jax.experimental.pallas import tpu_sc as plsc

N = 10000
D = 256
E = 160000
NC = 2
NS = 16
NW = NC * NS
L = 16

RPT = 320
LASTR = N - (NW - 1) * RPT
ACC_ROWS = RPT + 1
DUMP = RPT
CHUNK = 1600
NCH = E // CHUNK
U = 2
G = 64
CAP = CHUNK + G
NBMAX = CAP // G + 1


def _sc_body(x_hbm, src_hbm, dst_hbm, out_hbm,
             srcb0, dstb0, srcb1, dstb1, src_c, ldst_c,
             snap_src_a, snap_ld_a, snap_src_b, snap_ld_b,
             rows_a, rows_b, acc,
             gsem_a, gsem_b, esem0, esem1):
  c = lax.axis_index("c")
  s = lax.axis_index("s")
  w = c * NS + s
  lo = w * RPT
  hi = jnp.minimum(lo + RPT, N)
  zero_v = jnp.zeros((L,), jnp.float32)
  ones_i = jnp.ones((L,), jnp.int32)
  zeros_i = jnp.zeros((L,), jnp.int32)
  lane = lax.iota(jnp.int32, L)

  def _zrow(i, _):
    def _zcol(k, _):
      acc[i, pl.ds(k * L, L)] = zero_v
      return 0
    return lax.fori_loop(0, D // L, _zcol, 0)
  lax.fori_loop(0, ACC_ROWS, _zrow, 0)

  def fire(off, snap_src, snap_ld, rows, gsem):
    for k in range(G // L):
      snap_src[pl.ds(k * L, L)] = src_c[pl.ds(off + k * L, L)]
      snap_ld[pl.ds(k * L, L)] = ldst_c[pl.ds(off + k * L, L)]
    pltpu.async_copy(x_hbm.at[snap_src], rows, gsem)

  def process(snap_src, snap_ld, rows, gsem):
    pltpu.make_async_copy(x_hbm.at[snap_src], rows, gsem).wait()

    def _rowgrp(gg, _):
      ldv = snap_ld[pl.ds(gg * L, L)]
      for j in range(0, L, 2):
        ld0 = ldv[j]
        ld1 = ldv[j + 1]
        g0 = gg * L + j
        g1 = g0 + 1
        for k in range(D // L):
          v0 = rows[g0, pl.ds(k * L, L)]
          v1 = rows[g1, pl.ds(k * L, L)]
          plsc.addupdate(acc.at[ld0, pl.ds(k * L, L)], v0)
          plsc.addupdate(acc.at[ld1, pl.ds(k * L, L)], v1)
      return 0
    lax.fori_loop(0, G // L, _rowgrp, 0)

  SLOT_A = (snap_src_a, snap_ld_a, rows_a, gsem_a)
  SLOT_B = (snap_src_b, snap_ld_b, rows_b, gsem_b)

  def fire_process(off, bk):
    even = bk % 2 == 0

    @pl.when(even)
    def _():
      fire(off, *SLOT_A)

      @pl.when(bk >= 1)
      def _():
        process(*SLOT_B)

    @pl.when(jnp.logical_not(even))
    def _():
      fire(off, *SLOT_B)
      process(*SLOT_A)

  def start_chunk(ci, sb, db, sem):
    pltpu.async_copy(src_hbm.at[pl.ds(ci * CHUNK, CHUNK)], sb, sem)
    pltpu.async_copy(dst_hbm.at[pl.ds(ci * CHUNK, CHUNK)], db, sem)

  def wait_chunk(ci, sb, db, sem):
    pltpu.make_async_copy(src_hbm.at[pl.ds(ci * CHUNK, CHUNK)], sb, sem).wait()
    pltpu.make_async_copy(dst_hbm.at[pl.ds(ci * CHUNK, CHUNK)], db, sem).wait()

  def scan_chunk(sb, db, cur0):
    def _scan(i, cur):
      regs = []
      for u in range(U):
        sv = sb[pl.ds((i * U + u) * L, L)]
        dv = db[pl.ds((i * U + u) * L, L)]
        ld = dv - lo
        m = (ld >= 0) & (dv < hi)
        mi = jnp.where(m, ones_i, zeros_i)
        regs.append((sv, ld, m, plsc.cumsum(mi)))
      for sv, ld, m, cs in regs:
        pos = cur + cs - 1
        plsc.store_scatter(src_c, [pos], sv, mask=m)
        plsc.store_scatter(ldst_c, [pos], ld, mask=m)
        cur = cur + cs[L - 1]
      return cur
    return lax.fori_loop(0, CHUNK // (U * L), _scan, cur0)

  def drain(cur, bk):
    def _batch(b, bk):
      @pl.when((b + 1) * G <= cur)
      def _():
        fire_process(b * G, bk)
      return jnp.where((b + 1) * G <= cur, bk + 1, bk)
    bk = lax.fori_loop(0, NBMAX, _batch, bk)

    off0 = (cur // G) * G
    for k in range(G // L):
      sv = src_c[pl.ds(off0 + k * L, L)]
      lv = ldst_c[pl.ds(off0 + k * L, L)]
      src_c[pl.ds(k * L, L)] = sv
      ldst_c[pl.ds(k * L, L)] = lv
    return cur - off0, bk

  start_chunk(0, srcb0, dstb0, esem0)

  def _pair(ci2, carry):
    cur, bk = carry
    ci = ci2 * 2

    @pl.when(ci + 1 < NCH)
    def _():
      start_chunk(ci + 1, srcb1, dstb1, esem1)
    wait_chunk(ci, srcb0, dstb0, esem0)
    cur, bk = drain(scan_chunk(srcb0, dstb0, cur), bk)

    @pl.when(ci + 2 < NCH)
    def _():
      start_chunk(ci + 2, srcb0, dstb0, esem0)
    wait_chunk(ci + 1, srcb1, dstb1, esem1)
    cur, bk = drain(scan_chunk(srcb1, dstb1, cur), bk)
    return cur, bk

  rem, bk = lax.fori_loop(0, NCH // 2, _pair, (jnp.int32(0), jnp.int32(0)))

  @pl.when(rem > 0)
  def _():
    for k in range(G // L):
      pos = rem + (k * L) + lane
      plsc.store_scatter(src_c, [pos], zeros_i)
      plsc.store_scatter(ldst_c, [pos], jnp.full((L,), DUMP, jnp.int32))
    fire_process(0, bk)
  bk = jnp.where(rem > 0, bk + 1, bk)

  @pl.when(bk >= 1)
  def _():
    last_even = (bk - 1) % 2 == 0

    @pl.when(last_even)
    def _():
      process(*SLOT_A)

    @pl.when(jnp.logical_not(last_even))
    def _():
      process(*SLOT_B)

  @pl.when(w < NW - 1)
  def _():
    pltpu.sync_copy(acc.at[pl.ds(0, RPT)], out_hbm.at[pl.ds(lo, RPT)])

  @pl.when(w == NW - 1)
  def _():
    pltpu.sync_copy(acc.at[pl.ds(0, LASTR)], out_hbm.at[pl.ds(lo, LASTR)])


def _sc_agg(x, src, dst):
  mesh = plsc.VectorSubcoreMesh(core_axis_name="c", subcore_axis_name="s")
  kern = pl.kernel(
      _sc_body,
      out_type=jax.ShapeDtypeStruct((N, D), jnp.float32),
      mesh=mesh,
      compiler_params=pltpu.CompilerParams(needs_layout_passes=False),
      scratch_types=[
          pltpu.VMEM((CHUNK,), jnp.int32),
          pltpu.VMEM((CHUNK,), jnp.int32),
          pltpu.VMEM((CHUNK,), jnp.int32),
          pltpu.VMEM((CHUNK,), jnp.int32),
          pltpu.VMEM((CAP,), jnp.int32),
          pltpu.VMEM((CAP,), jnp.int32),
          pltpu.VMEM((G,), jnp.int32),
          pltpu.VMEM((G,), jnp.int32),
          pltpu.VMEM((G,), jnp.int32),
          pltpu.VMEM((G,), jnp.int32),
          pltpu.VMEM((G, D), jnp.float32),
          pltpu.VMEM((G, D), jnp.float32),
          pltpu.VMEM((ACC_ROWS, D), jnp.float32),
          pltpu.SemaphoreType.DMA,
          pltpu.SemaphoreType.DMA,
          pltpu.SemaphoreType.DMA,
          pltpu.SemaphoreType.DMA,
      ],
  )
  return kern(x, src, dst)


def _bf16x3_dot(a, b):
  ah = a.astype(jnp.bfloat16)
  al = (a - ah.astype(jnp.float32)).astype(jnp.bfloat16)
  bh = b.astype(jnp.bfloat16)
  bl = (b - bh.astype(jnp.float32)).astype(jnp.bfloat16)
  def d(x, y):
    return jnp.dot(x, y, preferred_element_type=jnp.float32)
  return d(ah, bh) + d(ah, bl) + d(al, bh)


def _dense_body(agg_ref, w1_ref, b1_ref, w2_ref, b2_ref, out_ref):
  h = jnp.maximum(_bf16x3_dot(agg_ref[...], w1_ref[...]) + b1_ref[...], 0.0)
  pooled = jnp.sum(h, axis=0, keepdims=True)
  out_ref[...] = _bf16x3_dot(pooled, w2_ref[...]) + b2_ref[...]


def _dense(agg, W1, b1, W2, b2):
  return pl.pallas_call(
      _dense_body,
      out_shape=jax.ShapeDtypeStruct((1, 1), jnp.float32),
  )(agg, W1, b1.reshape(1, D), W2, b2.reshape(1, 1))


@jax.jit
def kernel(x, edge_index, W1, b1, W2, b2):
  src = edge_index[0].astype(jnp.int32)
  dst = edge_index[1].astype(jnp.int32)
  agg = _sc_agg(x, src, dst)
  return _dense(agg, W1, b1, W2, b2)


# speedup vs baseline: 1.4464x; 1.1411x over previous
import jax
import jax.numpy as jnp
from jax import lax
from jax.experimental import pallas as pl
from jax.experimental.pallas import tpu as pltpu
from jax.experimental.pallas import tpu_sc as plsc

N = 10000
D = 256
E = 160000
NC = 2
NS = 16
NW = NC * NS
L = 16

RPT = 320
LASTR = N - (NW - 1) * RPT
ACC_ROWS = RPT + 1
DUMP = RPT
CHUNK = 1600
NCH = E // CHUNK
U = 2
G = 64
CAP = CHUNK + G
NBMAX = CAP // G + 1


def _sc_body(x_hbm, src_hbm, dst_hbm, out_hbm,
             srcb0, dstb0, srcb1, dstb1, src_c, ldst_c,
             snap_src_a, snap_ld_a, snap_src_b, snap_ld_b,
             rows_a, rows_b, acc,
             gsem_a, gsem_b, esem0, esem1):
  c = lax.axis_index("c")
  s = lax.axis_index("s")
  w = c * NS + s
  lo = w * RPT
  hi = jnp.minimum(lo + RPT, N)
  zero_v = jnp.zeros((L,), jnp.float32)
  ones_i = jnp.ones((L,), jnp.int32)
  zeros_i = jnp.zeros((L,), jnp.int32)
  lane = lax.iota(jnp.int32, L)

  def _zrow(i, _):
    def _zcol(k, _):
      acc[i, pl.ds(k * L, L)] = zero_v
      return 0
    return lax.fori_loop(0, D // L, _zcol, 0)
  lax.fori_loop(0, ACC_ROWS, _zrow, 0)

  def fire(off, snap_src, snap_ld, rows, gsem):
    for k in range(G // L):
      snap_src[pl.ds(k * L, L)] = src_c[pl.ds(off + k * L, L)]
      snap_ld[pl.ds(k * L, L)] = ldst_c[pl.ds(off + k * L, L)]
    pltpu.async_copy(x_hbm.at[snap_src], rows, gsem)

  def process(snap_src, snap_ld, rows, gsem):
    pltpu.make_async_copy(x_hbm.at[snap_src], rows, gsem).wait()

    def _rowgrp(gg, _):
      ldv = snap_ld[pl.ds(gg * L, L)]
      for j in range(0, L, 4):
        lds = [ldv[j + t] for t in range(4)]
        gs = [gg * L + j + t for t in range(4)]
        for k in range(D // L):
          vs = [rows[gs[t], pl.ds(k * L, L)] for t in range(4)]
          for t in range(4):
            plsc.addupdate(acc.at[lds[t], pl.ds(k * L, L)], vs[t])
      return 0
    lax.fori_loop(0, G // L, _rowgrp, 0)

  SLOT_A = (snap_src_a, snap_ld_a, rows_a, gsem_a)
  SLOT_B = (snap_src_b, snap_ld_b, rows_b, gsem_b)

  def fire_process(off, bk):
    even = bk % 2 == 0

    @pl.when(even)
    def _():
      fire(off, *SLOT_A)

      @pl.when(bk >= 1)
      def _():
        process(*SLOT_B)

    @pl.when(jnp.logical_not(even))
    def _():
      fire(off, *SLOT_B)
      process(*SLOT_A)

  def start_chunk(ci, sb, db, sem):
    pltpu.async_copy(src_hbm.at[pl.ds(ci * CHUNK, CHUNK)], sb, sem)
    pltpu.async_copy(dst_hbm.at[pl.ds(ci * CHUNK, CHUNK)], db, sem)

  def wait_chunk(ci, sb, db, sem):
    pltpu.make_async_copy(src_hbm.at[pl.ds(ci * CHUNK, CHUNK)], sb, sem).wait()
    pltpu.make_async_copy(dst_hbm.at[pl.ds(ci * CHUNK, CHUNK)], db, sem).wait()

  def scan_chunk(sb, db, cur0):
    def _scan(i, cur):
      regs = []
      for u in range(U):
        sv = sb[pl.ds((i * U + u) * L, L)]
        dv = db[pl.ds((i * U + u) * L, L)]
        ld = dv - lo
        m = (ld >= 0) & (dv < hi)
        mi = jnp.where(m, ones_i, zeros_i)
        regs.append((sv, ld, m, plsc.cumsum(mi)))
      for sv, ld, m, cs in regs:
        pos = cur + cs - 1
        plsc.store_scatter(src_c, [pos], sv, mask=m)
        plsc.store_scatter(ldst_c, [pos], ld, mask=m)
        cur = cur + cs[L - 1]
      return cur
    return lax.fori_loop(0, CHUNK // (U * L), _scan, cur0)

  def drain(cur, bk):
    def _batch(b, bk):
      @pl.when((b + 1) * G <= cur)
      def _():
        fire_process(b * G, bk)
      return jnp.where((b + 1) * G <= cur, bk + 1, bk)
    bk = lax.fori_loop(0, NBMAX, _batch, bk)

    off0 = (cur // G) * G
    for k in range(G // L):
      sv = src_c[pl.ds(off0 + k * L, L)]
      lv = ldst_c[pl.ds(off0 + k * L, L)]
      src_c[pl.ds(k * L, L)] = sv
      ldst_c[pl.ds(k * L, L)] = lv
    return cur - off0, bk

  start_chunk(0, srcb0, dstb0, esem0)

  def _pair(ci2, carry):
    cur, bk = carry
    ci = ci2 * 2

    @pl.when(ci + 1 < NCH)
    def _():
      start_chunk(ci + 1, srcb1, dstb1, esem1)
    wait_chunk(ci, srcb0, dstb0, esem0)
    cur, bk = drain(scan_chunk(srcb0, dstb0, cur), bk)

    @pl.when(ci + 2 < NCH)
    def _():
      start_chunk(ci + 2, srcb0, dstb0, esem0)
    wait_chunk(ci + 1, srcb1, dstb1, esem1)
    cur, bk = drain(scan_chunk(srcb1, dstb1, cur), bk)
    return cur, bk

  rem, bk = lax.fori_loop(0, NCH // 2, _pair, (jnp.int32(0), jnp.int32(0)))

  @pl.when(rem > 0)
  def _():
    for k in range(G // L):
      pos = rem + (k * L) + lane
      plsc.store_scatter(src_c, [pos], zeros_i)
      plsc.store_scatter(ldst_c, [pos], jnp.full((L,), DUMP, jnp.int32))
    fire_process(0, bk)
  bk = jnp.where(rem > 0, bk + 1, bk)

  @pl.when(bk >= 1)
  def _():
    last_even = (bk - 1) % 2 == 0

    @pl.when(last_even)
    def _():
      process(*SLOT_A)

    @pl.when(jnp.logical_not(last_even))
    def _():
      process(*SLOT_B)

  @pl.when(w < NW - 1)
  def _():
    pltpu.sync_copy(acc.at[pl.ds(0, RPT)], out_hbm.at[pl.ds(lo, RPT)])

  @pl.when(w == NW - 1)
  def _():
    pltpu.sync_copy(acc.at[pl.ds(0, LASTR)], out_hbm.at[pl.ds(lo, LASTR)])


def _sc_agg(x, src, dst):
  mesh = plsc.VectorSubcoreMesh(core_axis_name="c", subcore_axis_name="s")
  kern = pl.kernel(
      _sc_body,
      out_type=jax.ShapeDtypeStruct((N, D), jnp.float32),
      mesh=mesh,
      compiler_params=pltpu.CompilerParams(needs_layout_passes=False),
      scratch_types=[
          pltpu.VMEM((CHUNK,), jnp.int32),
          pltpu.VMEM((CHUNK,), jnp.int32),
          pltpu.VMEM((CHUNK,), jnp.int32),
          pltpu.VMEM((CHUNK,), jnp.int32),
          pltpu.VMEM((CAP,), jnp.int32),
          pltpu.VMEM((CAP,), jnp.int32),
          pltpu.VMEM((G,), jnp.int32),
          pltpu.VMEM((G,), jnp.int32),
          pltpu.VMEM((G,), jnp.int32),
          pltpu.VMEM((G,), jnp.int32),
          pltpu.VMEM((G, D), jnp.float32),
          pltpu.VMEM((G, D), jnp.float32),
          pltpu.VMEM((ACC_ROWS, D), jnp.float32),
          pltpu.SemaphoreType.DMA,
          pltpu.SemaphoreType.DMA,
          pltpu.SemaphoreType.DMA,
          pltpu.SemaphoreType.DMA,
      ],
  )
  return kern(x, src, dst)


def _bf16x3_dot(a, b):
  ah = a.astype(jnp.bfloat16)
  al = (a - ah.astype(jnp.float32)).astype(jnp.bfloat16)
  bh = b.astype(jnp.bfloat16)
  bl = (b - bh.astype(jnp.float32)).astype(jnp.bfloat16)
  def d(x, y):
    return jnp.dot(x, y, preferred_element_type=jnp.float32)
  return d(ah, bh) + d(ah, bl) + d(al, bh)


def _dense_body(agg_ref, w1_ref, b1_ref, w2_ref, b2_ref, out_ref):
  h = jnp.maximum(_bf16x3_dot(agg_ref[...], w1_ref[...]) + b1_ref[...], 0.0)
  pooled = jnp.sum(h, axis=0, keepdims=True)
  out_ref[...] = _bf16x3_dot(pooled, w2_ref[...]) + b2_ref[...]


def _dense(agg, W1, b1, W2, b2):
  return pl.pallas_call(
      _dense_body,
      out_shape=jax.ShapeDtypeStruct((1, 1), jnp.float32),
  )(agg, W1, b1.reshape(1, D), W2, b2.reshape(1, 1))


@jax.jit
def kernel(x, edge_index, W1, b1, W2, b2):
  src = edge_index[0].astype(jnp.int32)
  dst = edge_index[1].astype(jnp.int32)
  agg = _sc_agg(x, src, dst)
  return _dense(agg, W1, b1, W2, b2)


# speedup vs baseline: 1.4825x; 1.0250x over previous
import jax
import jax.numpy as jnp
from jax import lax
from jax.experimental import pallas as pl
from jax.experimental.pallas import tpu as pltpu
from jax.experimental.pallas import tpu_sc as plsc

N = 10000
D = 256
E = 160000
NC = 2
NS = 16
NW = NC * NS
L = 16

RPT = 320
LASTR = N - (NW - 1) * RPT
ACC_ROWS = RPT + 1
DUMP = RPT
CHUNK = 1600
NCH = E // CHUNK
U = 2
G = 64
CAP = CHUNK + G
NBMAX = CAP // G + 1


def _sc_body(x_hbm, src_hbm, dst_hbm, out_hbm,
             srcb0, dstb0, srcb1, dstb1, src_c, ldst_c,
             snap_src_a, snap_ld_a, snap_src_b, snap_ld_b,
             rows_a, rows_b, acc,
             gsem_a, gsem_b, esem0, esem1):
  c = lax.axis_index("c")
  s = lax.axis_index("s")
  w = c * NS + s
  lo = w * RPT
  hi = jnp.minimum(lo + RPT, N)
  zero_v = jnp.zeros((L,), jnp.float32)
  ones_i = jnp.ones((L,), jnp.int32)
  zeros_i = jnp.zeros((L,), jnp.int32)
  lane = lax.iota(jnp.int32, L)

  def _zrow(i, _):
    def _zcol(k, _):
      acc[i, pl.ds(k * L, L)] = zero_v
      return 0
    return lax.fori_loop(0, D // L, _zcol, 0)
  lax.fori_loop(0, ACC_ROWS, _zrow, 0)

  def fire(off, snap_src, snap_ld, rows, gsem):
    for k in range(G // L):
      snap_src[pl.ds(k * L, L)] = src_c[pl.ds(off + k * L, L)]
      snap_ld[pl.ds(k * L, L)] = ldst_c[pl.ds(off + k * L, L)]
    pltpu.async_copy(x_hbm.at[snap_src], rows, gsem)

  def process(snap_src, snap_ld, rows, gsem):
    pltpu.make_async_copy(x_hbm.at[snap_src], rows, gsem).wait()

    def _rowgrp(gg, _):
      ldv = snap_ld[pl.ds(gg * L, L)]
      for j in range(0, L, 8):
        lds = [ldv[j + t] for t in range(8)]
        gs = [gg * L + j + t for t in range(8)]
        for k in range(D // L):
          vs = [rows[gs[t], pl.ds(k * L, L)] for t in range(8)]
          for t in range(8):
            plsc.addupdate(acc.at[lds[t], pl.ds(k * L, L)], vs[t])
      return 0
    lax.fori_loop(0, G // L, _rowgrp, 0)

  SLOT_A = (snap_src_a, snap_ld_a, rows_a, gsem_a)
  SLOT_B = (snap_src_b, snap_ld_b, rows_b, gsem_b)

  def fire_process(off, bk):
    even = bk % 2 == 0

    @pl.when(even)
    def _():
      fire(off, *SLOT_A)

      @pl.when(bk >= 1)
      def _():
        process(*SLOT_B)

    @pl.when(jnp.logical_not(even))
    def _():
      fire(off, *SLOT_B)
      process(*SLOT_A)

  def start_chunk(ci, sb, db, sem):
    pltpu.async_copy(src_hbm.at[pl.ds(ci * CHUNK, CHUNK)], sb, sem)
    pltpu.async_copy(dst_hbm.at[pl.ds(ci * CHUNK, CHUNK)], db, sem)

  def wait_chunk(ci, sb, db, sem):
    pltpu.make_async_copy(src_hbm.at[pl.ds(ci * CHUNK, CHUNK)], sb, sem).wait()
    pltpu.make_async_copy(dst_hbm.at[pl.ds(ci * CHUNK, CHUNK)], db, sem).wait()

  def scan_chunk(sb, db, cur0):
    def _scan(i, cur):
      regs = []
      for u in range(U):
        sv = sb[pl.ds((i * U + u) * L, L)]
        dv = db[pl.ds((i * U + u) * L, L)]
        ld = dv - lo
        m = (ld >= 0) & (dv < hi)
        mi = jnp.where(m, ones_i, zeros_i)
        regs.append((sv, ld, m, plsc.cumsum(mi)))
      for sv, ld, m, cs in regs:
        pos = cur + cs - 1
        plsc.store_scatter(src_c, [pos], sv, mask=m)
        plsc.store_scatter(ldst_c, [pos], ld, mask=m)
        cur = cur + cs[L - 1]
      return cur
    return lax.fori_loop(0, CHUNK // (U * L), _scan, cur0)

  def drain(cur, bk):
    def _batch(b, bk):
      @pl.when((b + 1) * G <= cur)
      def _():
        fire_process(b * G, bk)
      return jnp.where((b + 1) * G <= cur, bk + 1, bk)
    bk = lax.fori_loop(0, NBMAX, _batch, bk)

    off0 = (cur // G) * G
    for k in range(G // L):
      sv = src_c[pl.ds(off0 + k * L, L)]
      lv = ldst_c[pl.ds(off0 + k * L, L)]
      src_c[pl.ds(k * L, L)] = sv
      ldst_c[pl.ds(k * L, L)] = lv
    return cur - off0, bk

  start_chunk(0, srcb0, dstb0, esem0)

  def _pair(ci2, carry):
    cur, bk = carry
    ci = ci2 * 2

    @pl.when(ci + 1 < NCH)
    def _():
      start_chunk(ci + 1, srcb1, dstb1, esem1)
    wait_chunk(ci, srcb0, dstb0, esem0)
    cur, bk = drain(scan_chunk(srcb0, dstb0, cur), bk)

    @pl.when(ci + 2 < NCH)
    def _():
      start_chunk(ci + 2, srcb0, dstb0, esem0)
    wait_chunk(ci + 1, srcb1, dstb1, esem1)
    cur, bk = drain(scan_chunk(srcb1, dstb1, cur), bk)
    return cur, bk

  rem, bk = lax.fori_loop(0, NCH // 2, _pair, (jnp.int32(0), jnp.int32(0)))

  @pl.when(rem > 0)
  def _():
    for k in range(G // L):
      pos = rem + (k * L) + lane
      plsc.store_scatter(src_c, [pos], zeros_i)
      plsc.store_scatter(ldst_c, [pos], jnp.full((L,), DUMP, jnp.int32))
    fire_process(0, bk)
  bk = jnp.where(rem > 0, bk + 1, bk)

  @pl.when(bk >= 1)
  def _():
    last_even = (bk - 1) % 2 == 0

    @pl.when(last_even)
    def _():
      process(*SLOT_A)

    @pl.when(jnp.logical_not(last_even))
    def _():
      process(*SLOT_B)

  @pl.when(w < NW - 1)
  def _():
    pltpu.sync_copy(acc.at[pl.ds(0, RPT)], out_hbm.at[pl.ds(lo, RPT)])

  @pl.when(w == NW - 1)
  def _():
    pltpu.sync_copy(acc.at[pl.ds(0, LASTR)], out_hbm.at[pl.ds(lo, LASTR)])


def _sc_agg(x, src, dst):
  mesh = plsc.VectorSubcoreMesh(core_axis_name="c", subcore_axis_name="s")
  kern = pl.kernel(
      _sc_body,
      out_type=jax.ShapeDtypeStruct((N, D), jnp.float32),
      mesh=mesh,
      compiler_params=pltpu.CompilerParams(needs_layout_passes=False),
      scratch_types=[
          pltpu.VMEM((CHUNK,), jnp.int32),
          pltpu.VMEM((CHUNK,), jnp.int32),
          pltpu.VMEM((CHUNK,), jnp.int32),
          pltpu.VMEM((CHUNK,), jnp.int32),
          pltpu.VMEM((CAP,), jnp.int32),
          pltpu.VMEM((CAP,), jnp.int32),
          pltpu.VMEM((G,), jnp.int32),
          pltpu.VMEM((G,), jnp.int32),
          pltpu.VMEM((G,), jnp.int32),
          pltpu.VMEM((G,), jnp.int32),
          pltpu.VMEM((G, D), jnp.float32),
          pltpu.VMEM((G, D), jnp.float32),
          pltpu.VMEM((ACC_ROWS, D), jnp.float32),
          pltpu.SemaphoreType.DMA,
          pltpu.SemaphoreType.DMA,
          pltpu.SemaphoreType.DMA,
          pltpu.SemaphoreType.DMA,
      ],
  )
  return kern(x, src, dst)


def _bf16x3_dot(a, b):
  ah = a.astype(jnp.bfloat16)
  al = (a - ah.astype(jnp.float32)).astype(jnp.bfloat16)
  bh = b.astype(jnp.bfloat16)
  bl = (b - bh.astype(jnp.float32)).astype(jnp.bfloat16)
  def d(x, y):
    return jnp.dot(x, y, preferred_element_type=jnp.float32)
  return d(ah, bh) + d(ah, bl) + d(al, bh)


def _dense_body(agg_ref, w1_ref, b1_ref, w2_ref, b2_ref, out_ref):
  h = jnp.maximum(_bf16x3_dot(agg_ref[...], w1_ref[...]) + b1_ref[...], 0.0)
  pooled = jnp.sum(h, axis=0, keepdims=True)
  out_ref[...] = _bf16x3_dot(pooled, w2_ref[...]) + b2_ref[...]


def _dense(agg, W1, b1, W2, b2):
  return pl.pallas_call(
      _dense_body,
      out_shape=jax.ShapeDtypeStruct((1, 1), jnp.float32),
  )(agg, W1, b1.reshape(1, D), W2, b2.reshape(1, 1))


@jax.jit
def kernel(x, edge_index, W1, b1, W2, b2):
  src = edge_index[0].astype(jnp.int32)
  dst = edge_index[1].astype(jnp.int32)
  agg = _sc_agg(x, src, dst)
  return _dense(agg, W1, b1, W2, b2)
